# baseline (speedup 1.0000x reference)
import functools

import jax
import jax.numpy as jnp
from jax import lax
from jax.experimental import pallas as pl
from jax.experimental.pallas import tpu as pltpu
from jax.experimental.pallas import tpu_sc as plsc

B = 16384
D = 32
NC = 2
NS = 16
NW = NC * NS
B_PER_W = B // NW


def _sc_gather_body(uidx_hbm, iidx_hbm, ug_hbm, ig_hbm, um_hbm, im_hbm,
                    out_ug, out_ig, out_um, out_im,
                    uidx_v, iidx_v, r_ug, r_ig, r_um, r_im, sem):
    wid = lax.axis_index("s") * NC + lax.axis_index("c")
    base = wid * B_PER_W
    pltpu.sync_copy(uidx_hbm.at[pl.ds(base, B_PER_W)], uidx_v)
    pltpu.sync_copy(iidx_hbm.at[pl.ds(base, B_PER_W)], iidx_v)
    c1 = pltpu.async_copy(ug_hbm.at[uidx_v], r_ug, sem)
    c2 = pltpu.async_copy(ig_hbm.at[iidx_v], r_ig, sem)
    c3 = pltpu.async_copy(um_hbm.at[uidx_v], r_um, sem)
    c4 = pltpu.async_copy(im_hbm.at[iidx_v], r_im, sem)
    c1.wait(); c2.wait(); c3.wait(); c4.wait()
    pltpu.sync_copy(r_ug, out_ug.at[pl.ds(base, B_PER_W)])
    pltpu.sync_copy(r_ig, out_ig.at[pl.ds(base, B_PER_W)])
    pltpu.sync_copy(r_um, out_um.at[pl.ds(base, B_PER_W)])
    pltpu.sync_copy(r_im, out_im.at[pl.ds(base, B_PER_W)])


_sc_gather = functools.partial(
    pl.kernel,
    out_type=[jax.ShapeDtypeStruct((B, D), jnp.float32)] * 4,
    mesh=plsc.VectorSubcoreMesh(core_axis_name="c", subcore_axis_name="s"),
    compiler_params=pltpu.CompilerParams(use_tc_tiling_on_sc=False),
    scratch_types=[
        pltpu.VMEM((B_PER_W,), jnp.int32),
        pltpu.VMEM((B_PER_W,), jnp.int32),
        pltpu.VMEM((B_PER_W, D), jnp.float32),
        pltpu.VMEM((B_PER_W, D), jnp.float32),
        pltpu.VMEM((B_PER_W, D), jnp.float32),
        pltpu.VMEM((B_PER_W, D), jnp.float32),
        pltpu.SemaphoreType.DMA,
    ],
)(_sc_gather_body)


def _dot_t(x, w):
    return lax.dot_general(x, w, (((1,), (1,)), ((), ())),
                           preferred_element_type=jnp.float32)


def _tc_dense_body(ug_ref, ig_ref, um_ref, im_ref,
                   w1a_ref, w1b_ref, b1_ref, w2_ref, b2_ref, w3_ref, b3_ref,
                   wpa_ref, wpb_ref, bp_ref, out_ref):
    mf = ug_ref[...] * ig_ref[...]
    h = _dot_t(um_ref[...], w1a_ref[...]) + _dot_t(im_ref[...], w1b_ref[...])
    h = jnp.maximum(h + b1_ref[...], 0.0)
    h = jnp.maximum(_dot_t(h, w2_ref[...]) + b2_ref[...], 0.0)
    h = jnp.maximum(_dot_t(h, w3_ref[...]) + b3_ref[...], 0.0)
    out_ref[...] = _dot_t(mf, wpa_ref[...]) + _dot_t(h, wpb_ref[...]) + bp_ref[...]


def kernel(user_indices, item_indices, U_gmf, I_gmf, U_mlp, I_mlp,
           W1, b1, W2, b2, W3, b3, Wp, bp):
    ug, ig, um, im = _sc_gather(user_indices, item_indices,
                                U_gmf, I_gmf, U_mlp, I_mlp)
    w1a, w1b = W1[:, :D], W1[:, D:]
    wpa, wpb = Wp[:, :D], Wp[:, D:]
    pred = pl.pallas_call(
        _tc_dense_body,
        out_shape=jax.ShapeDtypeStruct((B, 1), jnp.float32),
    )(ug, ig, um, im,
      w1a, w1b, b1.reshape(1, -1), W2, b2.reshape(1, -1),
      W3, b3.reshape(1, -1), wpa, wpb, bp.reshape(1, 1))
    return pred.reshape(-1)
